# Initial kernel scaffold; baseline (speedup 1.0000x reference)
import jax
import jax.numpy as jnp
from jax.experimental import pallas as pl


def kernel(boxes, scores, post_nms_top_n):
    raise NotImplementedError("write your pallas kernel here")


# speedup vs baseline: 111.6176x; 111.6176x over previous
import functools

import jax
import jax.numpy as jnp
from jax.experimental import pallas as pl

N = 5000
NPAD = 5120
B = 128
NB = NPAD // B
TOP = 1000
TH = 0.7


def _sup_block(px1, py1, px2, py2, pa, cx1, cy1, cx2, cy2, ca):
    xx1 = jnp.maximum(px1, cx1)
    yy1 = jnp.maximum(py1, cy1)
    xx2 = jnp.minimum(px2, cx2)
    yy2 = jnp.minimum(py2, cy2)
    inter = jnp.maximum(xx2 - xx1, 0.0) * jnp.maximum(yy2 - yy1, 0.0)
    denom = pa + ca - inter + 1e-9
    return (inter > TH * denom).astype(jnp.float32)


def _row0(v):
    rmask = (jax.lax.broadcasted_iota(jnp.int32, (8, B), 0) == 0)
    return jnp.broadcast_to(v, (8, B)) * rmask.astype(jnp.float32)


def _nms_body(x1r, y1r, x2r, y2r, ar, x1c, y1c, x2c, y2c, ac, keep_ref):
    keep_ref[...] = jnp.zeros((NB, 1, B), jnp.float32)
    lane = jax.lax.broadcasted_iota(jnp.int32, (1, B), 1)
    tri = (jax.lax.broadcasted_iota(jnp.int32, (B, B), 0)
           < jax.lax.broadcasted_iota(jnp.int32, (B, B), 1)).astype(jnp.float32)

    def row(ref, j):
        return ref[pl.ds(j, 1), 0, :]

    def colblk(ref, p):
        return ref[pl.ds(pl.multiple_of(p * B, B), B), :]

    def mm(k_row, s):
        out = jax.lax.dot_general(_row0(k_row), s, (((1,), (0,)), ((), ())),
                                  preferred_element_type=jnp.float32)
        return out[0:1, :]

    def blk_body(state):
        j, kept = state
        cx1, cy1, cx2, cy2, car = (row(x1r, j), row(y1r, j), row(x2r, j),
                                   row(y2r, j), row(ar, j))

        def pbody(p, acc):
            s = _sup_block(colblk(x1c, p), colblk(y1c, p), colblk(x2c, p),
                           colblk(y2c, p), colblk(ac, p),
                           cx1, cy1, cx2, cy2, car)
            kprev = keep_ref[pl.ds(p, 1), 0, :]
            return acc + mm(kprev, s)

        acc = jax.lax.fori_loop(0, j, pbody, jnp.zeros((1, B), jnp.float32))
        valid = (j * B + lane) < N
        incoming = jnp.where((acc == 0.0) & valid, 1.0, 0.0)

        scc = _sup_block(colblk(x1c, j), colblk(y1c, j), colblk(x2c, j),
                         colblk(y2c, j), colblk(ac, j),
                         cx1, cy1, cx2, cy2, car) * tri

        def fcond(s):
            return s[1]

        def fbody(s):
            k, _ = s
            hit = mm(k, scc)
            new = jnp.where(hit == 0.0, incoming, 0.0)
            return new, jnp.any(new != k)

        keep_j, _ = jax.lax.while_loop(fcond, fbody,
                                       (incoming, jnp.array(True)))
        keep_ref[pl.ds(j, 1), 0, :] = keep_j
        return j + 1, kept + jnp.sum(keep_j)

    def blk_cond(state):
        j, kept = state
        return (j < NB) & (kept < float(TOP))

    jax.lax.while_loop(blk_cond, blk_body, (jnp.int32(0), jnp.float32(0.0)))


@jax.jit
def _nms_keep(bs):
    x1, y1, x2, y2 = bs[:, 0], bs[:, 1], bs[:, 2], bs[:, 3]
    areas = (x2 - x1) * (y2 - y1)
    rows = [v.reshape(NB, 1, B) for v in (x1, y1, x2, y2, areas)]
    cols = [v.reshape(NPAD, 1) for v in (x1, y1, x2, y2, areas)]
    keep = pl.pallas_call(
        _nms_body,
        out_shape=jax.ShapeDtypeStruct((NB, 1, B), jnp.float32),
    )(*rows, *cols)
    return keep.reshape(NPAD)


def kernel(boxes, scores, post_nms_top_n):
    order = jnp.argsort(-scores)
    bs = jnp.take(boxes, order, axis=0)
    ss = jnp.take(scores, order)
    bpad = jnp.pad(bs, ((0, NPAD - N), (0, 0)))
    keep = _nms_keep(bpad)[:N] > 0.0

    idx = jnp.arange(N)
    priority = jnp.where(keep, idx, N + idx)
    sel = jnp.argsort(priority)[:TOP]
    rois = jnp.concatenate([jnp.take(ss, sel)[:, None],
                            jnp.take(bs, sel, axis=0)], axis=1)
    return rois
